# baseline (speedup 1.0000x reference)
import jax
import jax.numpy as jnp
from jax.experimental import pallas as pl

_K = 16
_SHIFT = _K - 1


def _span_kernel(x_cur_ref, x_nxt_ref, out_ref, *, nb):
    i = pl.program_id(1)
    cur = x_cur_ref[0]
    nxt = x_nxt_ref[0]
    shifted = jnp.concatenate([cur[_SHIFT:], nxt[:_SHIFT]], axis=0)
    out_ref[0, :, 0] = cur
    out_ref[0, :, 1] = shifted

    @pl.when(i == nb - 1)
    def _():
        out_ref[0, pl.ds(out_ref.shape[1] - _SHIFT, _SHIFT), 1] = jnp.zeros(
            (_SHIFT,) + cur.shape[1:], cur.dtype
        )


def kernel(x):
    B, L, D = x.shape
    TL = 1024
    nb = L // TL
    x4 = x.reshape(B, L, 8, D // 8)

    out = pl.pallas_call(
        lambda a, b, o: _span_kernel(a, b, o, nb=nb),
        grid=(B, nb),
        in_specs=[
            pl.BlockSpec((1, TL, 8, D // 8), lambda b, i: (b, i, 0, 0)),
            pl.BlockSpec(
                (1, 16, 8, D // 8),
                lambda b, i: (b, jnp.minimum((i + 1) * (TL // 16), L // 16 - 1), 0, 0),
            ),
        ],
        out_specs=pl.BlockSpec((1, TL, 2, 8, D // 8), lambda b, i: (b, i, 0, 0, 0)),
        out_shape=jax.ShapeDtypeStruct((B, L, 2, 8, D // 8), x.dtype),
    )(x4, x4)
    return out.reshape(B, L, 2, D)


# speedup vs baseline: 2.9520x; 2.9520x over previous
import functools

import jax
import jax.numpy as jnp
from jax import lax
from jax.experimental import pallas as pl
from jax.experimental.pallas import tpu as pltpu
from jax.experimental.pallas import tpu_sc as plsc

_K = 16
_SHIFT = _K - 1
_NC, _NS = 2, 16
_CHUNK = 64


def kernel(x):
    B, L, D = x.shape
    nw = _NC * _NS
    rows_per_w = (B * L) // nw
    workers_per_b = L // rows_per_w
    nchunks = rows_per_w // _CHUNK

    x2 = x.reshape(B * L, D)
    mesh = plsc.VectorSubcoreMesh(core_axis_name="c", subcore_axis_name="s")

    @functools.partial(
        pl.kernel,
        mesh=mesh,
        out_type=jax.ShapeDtypeStruct((B, L, 2, D), x.dtype),
        scratch_types=[
            pltpu.VMEM((_CHUNK, D), x.dtype),
            pltpu.VMEM((_CHUNK,), jnp.int32),
            pltpu.VMEM((_SHIFT, D), x.dtype),
            pltpu.SemaphoreType.DMA,
        ],
    )
    def span_sc(x_hbm, out_hbm, buf, idx_v, zbuf, sem):
        wid = lax.axis_index("s") * _NC + lax.axis_index("c")
        b = wid // workers_per_b
        s = (wid % workers_per_b) * rows_per_w

        def chunk_body(g, carry):
            r = s + g * _CHUNK
            pltpu.sync_copy(x_hbm.at[pl.ds(b * L + r, _CHUNK)], buf)
            pltpu.sync_copy(buf, out_hbm.at[b, pl.ds(r, _CHUNK), 0])

            first = r == 0

            @pl.when(first)
            def _():
                for j in range(_CHUNK // 16):
                    idx_v[pl.ds(j * 16, 16)] = (
                        lax.iota(jnp.int32, 16) + (b * L + _SHIFT + j * 16)
                    )
                pltpu.async_copy(x_hbm.at[idx_v], buf, sem).wait()
                pltpu.sync_copy(buf, out_hbm.at[b, pl.ds(0, _CHUNK), 1])

            @pl.when(jnp.logical_not(first))
            def _():
                pltpu.sync_copy(
                    buf, out_hbm.at[b, pl.ds(r - _SHIFT, _CHUNK), 1]
                )

            return carry

        lax.fori_loop(0, nchunks, chunk_body, 0)

        @pl.when(wid % workers_per_b == workers_per_b - 1)
        def _():
            zero = jnp.zeros((16,), x.dtype)

            def zrow(i, carry):
                zbuf[i // (D // 16), pl.ds((i % (D // 16)) * 16, 16)] = zero
                return carry

            lax.fori_loop(0, (_SHIFT * D) // 16, zrow, 0)
            pltpu.sync_copy(zbuf, out_hbm.at[b, pl.ds(L - _SHIFT, _SHIFT), 1])

    return span_sc(x2)
